# baseline (speedup 1.0000x reference)
import jax
import jax.numpy as jnp
from jax.experimental import pallas as pl
from jax.experimental.pallas import tpu as pltpu


def _dot(x, w):
    return jnp.dot(x, w, preferred_element_type=jnp.float32)


def _dot_t(x, w):
    return jax.lax.dot_general(
        x, w, (((1,), (1,)), ((), ())), preferred_element_type=jnp.float32)


def _seq_gru_kernel(h_ref, xo_ref, fo_ref, x_ref, m_ref,
                    w1_ref, b1_ref, w2_ref, b2_ref,
                    wih_ref, whh_ref, bih_ref, bhh_ref,
                    wp_ref, bp_ref,
                    hout_ref, loss_ref, lpre_ref,
                    h_scr, wcat_scr, w2t_scr):
    B, D = x_ref.shape
    H = w1_ref.shape[0]
    T = xo_ref.shape[0]
    P = bp_ref.shape[1]

    h_scr[...] = h_ref[:B, :]
    wcat_scr[:H, :] = jnp.transpose(w1_ref[...])
    wcat_scr[H:, :] = whh_ref[...]
    wcat = wcat_scr[...]
    w2t_scr[...] = jnp.transpose(w2_ref[...])
    w2t = jnp.concatenate([w2t_scr[:D, :], w2t_scr[D:, :]], axis=1)
    b1 = b1_ref[...]
    b2 = b2_ref[...]
    wih = wih_ref[...]
    bih = bih_ref[...]
    bhh = bhh_ref[...]
    wpb = jnp.concatenate([w2t, wp_ref[...], bp_ref[...]], axis=1)
    xm = jnp.transpose(xo_ref[...])
    fm = jnp.transpose(fo_ref[...])

    iota_d = jax.lax.broadcasted_iota(jnp.int32, (B, D), 1)
    iota_t = jax.lax.broadcasted_iota(jnp.int32, (B, T), 1)
    zD = jnp.zeros((B, D), jnp.float32)

    def step(t, hidden):
        colm = iota_t == t
        xs = jnp.sum(jnp.where(colm, xm, 0.0), axis=1, keepdims=True)
        fs = jnp.sum(jnp.where(colm, fm, 0), axis=1, keepdims=True)
        ohm = iota_d == fs
        onehot = ohm.astype(jnp.float32)
        sel = _dot(onehot, wpb)
        w2sel = sel[:, :2 * H]
        wsel = sel[:, 2 * H:]
        b2_f = jnp.sum(jnp.where(ohm, b2[:1, :], zD), axis=1, keepdims=True)
        b2_fl = jnp.sum(jnp.where(ohm, b2[1:, :], zD), axis=1, keepdims=True)

        hw = _dot_t(hidden, wcat)
        a = jnp.maximum(hw[:, :H] + b1, 0.0)
        gh = hw[:, H:] + bhh
        mean_f = jnp.sum(a * w2sel[:, :H], axis=1, keepdims=True) + b2_f
        logvar_f = jnp.sum(a * w2sel[:, H:], axis=1, keepdims=True) + b2_fl
        sigma = jnp.exp(0.5 * logvar_f)
        err = (xs - mean_f) / sigma
        dloss = 0.5 * jnp.sum(err * err + logvar_f)

        gru_in = jnp.maximum(
            xs * wsel[:, :P] + mean_f * wsel[:, P:2 * P]
            + logvar_f * wsel[:, 2 * P:3 * P] + err * wsel[:, 3 * P:4 * P]
            + wsel[:, 4 * P:], 0.0)
        gi = _dot_t(gru_in, wih) + bih
        r = jax.nn.sigmoid(gi[:, :H] + gh[:, :H])
        z = jax.nn.sigmoid(gi[:, H:2 * H] + gh[:, H:2 * H])
        n = jnp.tanh(gi[:, 2 * H:] + r * gh[:, 2 * H:])
        h_new = (1.0 - z) * n + z * hidden
        return h_new, dloss

    hidden = h_scr[...]
    hw0 = _dot_t(hidden, wcat)
    a0 = jnp.maximum(hw0[:, :H] + b1, 0.0)
    gh0 = hw0[:, H:] + bhh
    p0 = _dot(a0, w2_ref[...])
    mean0 = p0[:, :D] + b2[:1, :]
    logvar0 = p0[:, D:] + b2[1:, :]
    sigma0 = jnp.exp(0.5 * logvar0)
    e0 = (x_ref[...] - mean0) / sigma0
    lpre_ref[...] = 0.5 * ((e0 * e0 + logvar0) * m_ref[...])

    colm0 = iota_t == 0
    xs0 = jnp.sum(jnp.where(colm0, xm, 0.0), axis=1, keepdims=True)
    fs0 = jnp.sum(jnp.where(colm0, fm, 0), axis=1, keepdims=True)
    ohm0 = iota_d == fs0
    onehot0 = ohm0.astype(jnp.float32)
    wsel0 = _dot(onehot0, wpb)[:, 2 * H:]
    mean_f0 = jnp.sum(jnp.where(ohm0, mean0, zD), axis=1, keepdims=True)
    logvar_f0 = jnp.sum(jnp.where(ohm0, logvar0, zD), axis=1, keepdims=True)
    sg0 = jnp.exp(0.5 * logvar_f0)
    err0 = (xs0 - mean_f0) / sg0
    loss = 0.5 * jnp.sum(err0 * err0 + logvar_f0)
    gru_in0 = jnp.maximum(
        xs0 * wsel0[:, :P] + mean_f0 * wsel0[:, P:2 * P]
        + logvar_f0 * wsel0[:, 2 * P:3 * P] + err0 * wsel0[:, 3 * P:4 * P]
        + wsel0[:, 4 * P:], 0.0)
    gi0 = _dot_t(gru_in0, wih) + bih
    r0 = jax.nn.sigmoid(gi0[:, :H] + gh0[:, :H])
    z0 = jax.nn.sigmoid(gi0[:, H:2 * H] + gh0[:, H:2 * H])
    n0 = jnp.tanh(gi0[:, 2 * H:] + r0 * gh0[:, 2 * H:])
    h_scr[...] = (1.0 - z0) * n0 + z0 * hidden

    def body(t, loss):
        h_new, dloss = step(t, h_scr[...])
        h_scr[...] = h_new
        return loss + dloss

    loss = jax.lax.fori_loop(1, T, body, loss, unroll=16)
    loss_ref[0, 0] = loss
    hout_ref[:B, :] = h_scr[...]
    hout_ref[B:, :] = h_ref[B:, :]


def kernel(h, X_obs_data, F_obs_data, i_obs, X, M, W1, b1, W2, b2,
           W_ih, W_hh, b_ih, b_hh, w_prep, bias_prep):
    B = i_obs.shape[0]
    T = X_obs_data.shape[0] // B
    N = h.shape[0]
    D = X.shape[1]
    H = h.shape[1]
    P = w_prep.shape[2]

    out_shapes = (
        jax.ShapeDtypeStruct((N, H), jnp.float32),
        jax.ShapeDtypeStruct((1, 1), jnp.float32),
        jax.ShapeDtypeStruct((B, D), jnp.float32),
    )
    h2, loss, lpre = pl.pallas_call(
        _seq_gru_kernel,
        out_shape=out_shapes,
        out_specs=(
            pl.BlockSpec(memory_space=pltpu.VMEM),
            pl.BlockSpec(memory_space=pltpu.SMEM),
            pl.BlockSpec(memory_space=pltpu.VMEM),
        ),
        scratch_shapes=[pltpu.VMEM((B, H), jnp.float32),
                        pltpu.VMEM((4 * H, H), jnp.float32),
                        pltpu.VMEM((2 * D, H), jnp.float32)],
    )(h, X_obs_data.reshape(T, B), F_obs_data.reshape(T, B).astype(jnp.int32),
      X, M, W1, b1, W2, b2.reshape(2, D),
      W_ih, W_hh, b_ih, b_hh, w_prep.reshape(D, 4 * P), bias_prep)

    return (h2, loss[0, 0], lpre)


# speedup vs baseline: 1.0242x; 1.0242x over previous
import jax
import jax.numpy as jnp
from jax.experimental import pallas as pl
from jax.experimental.pallas import tpu as pltpu


def _dot(x, w):
    return jnp.dot(x, w, preferred_element_type=jnp.float32)


def _dot_t(x, w):
    return jax.lax.dot_general(
        x, w, (((1,), (1,)), ((), ())), preferred_element_type=jnp.float32)


def _seq_gru_kernel(h_ref, xo_ref, fo_ref, x_ref, m_ref,
                    w1_ref, b1_ref, w2_ref, b2_ref,
                    wih_ref, whh_ref, bih_ref, bhh_ref,
                    wp_ref, bp_ref,
                    hout_ref, loss_ref, lpre_ref,
                    h_scr, wcat_scr, w2t_scr):
    B, D = x_ref.shape
    H = w1_ref.shape[0]
    T = xo_ref.shape[0]
    P = bp_ref.shape[1]

    h_scr[...] = h_ref[:B, :]
    wcat_scr[:H, :] = jnp.transpose(w1_ref[...])
    wcat_scr[H:, :] = whh_ref[...]
    wcat = wcat_scr[...]
    w2t_scr[...] = jnp.transpose(w2_ref[...])
    w2t = jnp.concatenate([w2t_scr[:D, :], w2t_scr[D:, :]], axis=1)
    b1 = b1_ref[...]
    b2 = b2_ref[...]
    wih = wih_ref[...]
    bih = bih_ref[...]
    bhh = bhh_ref[...]
    wpb = jnp.concatenate([w2t, wp_ref[...], bp_ref[...]], axis=1)
    xm = jnp.transpose(xo_ref[...])
    fm = jnp.transpose(fo_ref[...])

    iota_d = jax.lax.broadcasted_iota(jnp.int32, (B, D), 1)
    iota_t = jax.lax.broadcasted_iota(jnp.int32, (B, T), 1)
    zD = jnp.zeros((B, D), jnp.float32)

    def step(t, hidden):
        colm = iota_t == t
        xs = jnp.sum(jnp.where(colm, xm, 0.0), axis=1, keepdims=True)
        fs = jnp.sum(jnp.where(colm, fm, 0), axis=1, keepdims=True)
        ohm = iota_d == fs
        onehot = ohm.astype(jnp.float32)
        sel = _dot(onehot, wpb)
        w2sel = sel[:, :2 * H]
        wsel = sel[:, 2 * H:]
        b2_f = jnp.sum(jnp.where(ohm, b2[:1, :], zD), axis=1, keepdims=True)
        b2_fl = jnp.sum(jnp.where(ohm, b2[1:, :], zD), axis=1, keepdims=True)

        hw = _dot_t(hidden, wcat)
        a = jnp.maximum(hw[:, :H] + b1, 0.0)
        gh = hw[:, H:] + bhh
        mean_f = jnp.sum(a * w2sel[:, :H], axis=1, keepdims=True) + b2_f
        logvar_f = jnp.sum(a * w2sel[:, H:], axis=1, keepdims=True) + b2_fl
        sigma = jnp.exp(0.5 * logvar_f)
        err = (xs - mean_f) / sigma
        dloss = 0.5 * jnp.sum(err * err + logvar_f)

        gru_in = jnp.maximum(
            xs * wsel[:, :P] + mean_f * wsel[:, P:2 * P]
            + logvar_f * wsel[:, 2 * P:3 * P] + err * wsel[:, 3 * P:4 * P]
            + wsel[:, 4 * P:], 0.0)
        gi = _dot_t(gru_in, wih) + bih
        r = jax.nn.sigmoid(gi[:, :H] + gh[:, :H])
        z = jax.nn.sigmoid(gi[:, H:2 * H] + gh[:, H:2 * H])
        n = jnp.tanh(gi[:, 2 * H:] + r * gh[:, 2 * H:])
        h_new = (1.0 - z) * n + z * hidden
        return h_new, dloss

    hidden = h_scr[...]
    hw0 = _dot_t(hidden, wcat)
    a0 = jnp.maximum(hw0[:, :H] + b1, 0.0)
    gh0 = hw0[:, H:] + bhh
    p0 = _dot(a0, w2_ref[...])
    mean0 = p0[:, :D] + b2[:1, :]
    logvar0 = p0[:, D:] + b2[1:, :]
    sigma0 = jnp.exp(0.5 * logvar0)
    e0 = (x_ref[...] - mean0) / sigma0
    lpre_ref[...] = 0.5 * ((e0 * e0 + logvar0) * m_ref[...])

    colm0 = iota_t == 0
    xs0 = jnp.sum(jnp.where(colm0, xm, 0.0), axis=1, keepdims=True)
    fs0 = jnp.sum(jnp.where(colm0, fm, 0), axis=1, keepdims=True)
    ohm0 = iota_d == fs0
    onehot0 = ohm0.astype(jnp.float32)
    wsel0 = _dot(onehot0, wpb)[:, 2 * H:]
    mean_f0 = jnp.sum(jnp.where(ohm0, mean0, zD), axis=1, keepdims=True)
    logvar_f0 = jnp.sum(jnp.where(ohm0, logvar0, zD), axis=1, keepdims=True)
    sg0 = jnp.exp(0.5 * logvar_f0)
    err0 = (xs0 - mean_f0) / sg0
    loss = 0.5 * jnp.sum(err0 * err0 + logvar_f0)
    gru_in0 = jnp.maximum(
        xs0 * wsel0[:, :P] + mean_f0 * wsel0[:, P:2 * P]
        + logvar_f0 * wsel0[:, 2 * P:3 * P] + err0 * wsel0[:, 3 * P:4 * P]
        + wsel0[:, 4 * P:], 0.0)
    gi0 = _dot_t(gru_in0, wih) + bih
    r0 = jax.nn.sigmoid(gi0[:, :H] + gh0[:, :H])
    z0 = jax.nn.sigmoid(gi0[:, H:2 * H] + gh0[:, H:2 * H])
    n0 = jnp.tanh(gi0[:, 2 * H:] + r0 * gh0[:, 2 * H:])
    h_scr[...] = (1.0 - z0) * n0 + z0 * hidden

    def body(t, loss):
        h_new, dloss = step(t, h_scr[...])
        h_scr[...] = h_new
        return loss + dloss

    loss = jax.lax.fori_loop(1, T, body, loss, unroll=8)
    loss_ref[0, 0] = loss
    hout_ref[:B, :] = h_scr[...]
    hout_ref[B:, :] = h_ref[B:, :]


def kernel(h, X_obs_data, F_obs_data, i_obs, X, M, W1, b1, W2, b2,
           W_ih, W_hh, b_ih, b_hh, w_prep, bias_prep):
    B = i_obs.shape[0]
    T = X_obs_data.shape[0] // B
    N = h.shape[0]
    D = X.shape[1]
    H = h.shape[1]
    P = w_prep.shape[2]

    out_shapes = (
        jax.ShapeDtypeStruct((N, H), jnp.float32),
        jax.ShapeDtypeStruct((1, 1), jnp.float32),
        jax.ShapeDtypeStruct((B, D), jnp.float32),
    )
    h2, loss, lpre = pl.pallas_call(
        _seq_gru_kernel,
        out_shape=out_shapes,
        out_specs=(
            pl.BlockSpec(memory_space=pltpu.VMEM),
            pl.BlockSpec(memory_space=pltpu.SMEM),
            pl.BlockSpec(memory_space=pltpu.VMEM),
        ),
        scratch_shapes=[pltpu.VMEM((B, H), jnp.float32),
                        pltpu.VMEM((4 * H, H), jnp.float32),
                        pltpu.VMEM((2 * D, H), jnp.float32)],
    )(h, X_obs_data.reshape(T, B), F_obs_data.reshape(T, B).astype(jnp.int32),
      X, M, W1, b1, W2, b2.reshape(2, D),
      W_ih, W_hh, b_ih, b_hh, w_prep.reshape(D, 4 * P), bias_prep)

    return (h2, loss[0, 0], lpre)
